# baseline (speedup 1.0000x reference)
import functools
import math

import jax
import jax.numpy as jnp
from jax import lax
from jax.experimental import pallas as pl
from jax.experimental.pallas import tpu as pltpu
from jax.experimental.pallas import tpu_sc as plsc

_EMB = 64
_SCALE = math.sqrt(_EMB)
_LANES = 16
_IDX_PER_STREAM = 128


def _emb_kernel_body(n_per_w, chunk, num_cores, tokens_hbm, table_hbm, out_hbm,
                     idx_v, rows_v, gsem):
    k = chunk // _IDX_PER_STREAM
    n_chunks = n_per_w // chunk
    wid = lax.axis_index("s") * num_cores + lax.axis_index("c")
    base = wid * n_per_w
    base_g = wid * (n_per_w // _IDX_PER_STREAM)

    @pl.loop(0, n_chunks)
    def _chunk_loop(c):
        row0 = base + c * chunk
        pltpu.sync_copy(tokens_hbm.at[pl.ds(base_g + c * k, k)], idx_v)
        for j in range(k):
            pltpu.async_copy(
                table_hbm.at[idx_v.at[j]],
                rows_v.at[pl.ds(j * _IDX_PER_STREAM, _IDX_PER_STREAM)],
                gsem,
            )
        for j in range(k):
            pltpu.make_async_copy(
                table_hbm.at[idx_v.at[j]],
                rows_v.at[pl.ds(j * _IDX_PER_STREAM, _IDX_PER_STREAM)],
                gsem,
            ).wait()

        @plsc.parallel_loop(0, chunk, unroll=4)
        def _scale(r):
            for d in range(_EMB // _LANES):
                sl = (r, pl.ds(d * _LANES, _LANES))
                rows_v[sl] = rows_v[sl] * _SCALE

        pltpu.sync_copy(rows_v, out_hbm.at[pl.ds(row0, chunk)])


def kernel(tokens, table):
    b, l = tokens.shape
    v, d = table.shape
    n = b * l
    info = plsc.get_sparse_core_info()
    nw = info.num_cores * info.num_subcores
    n_per_w = n // nw
    chunk = 512

    mesh = plsc.VectorSubcoreMesh(core_axis_name="c", subcore_axis_name="s")
    emb = pl.kernel(
        functools.partial(_emb_kernel_body, n_per_w, chunk, info.num_cores),
        out_type=jax.ShapeDtypeStruct((n, d), jnp.float32),
        mesh=mesh,
        scratch_types=[
            pltpu.VMEM((chunk // _IDX_PER_STREAM, _IDX_PER_STREAM), jnp.int32),
            pltpu.VMEM((chunk, d), jnp.float32),
            pltpu.SemaphoreType.DMA,
        ],
        compiler_params=pltpu.CompilerParams(use_tc_tiling_on_sc=False),
    )
    flat = emb(jnp.reshape(tokens, (n // _IDX_PER_STREAM, _IDX_PER_STREAM)), table)
    return jnp.reshape(flat, (b, l, d))


# speedup vs baseline: 1.0803x; 1.0803x over previous
import functools
import math

import jax
import jax.numpy as jnp
from jax import lax
from jax.experimental import pallas as pl
from jax.experimental.pallas import tpu as pltpu
from jax.experimental.pallas import tpu_sc as plsc

_EMB = 64
_SCALE = math.sqrt(_EMB)
_LANES = 16
_IDX_PER_STREAM = 128
_CHUNK = 640
_NBUF = 2


def _emb_kernel_body(n_per_w, num_cores, tokens_hbm, table_hbm, out_hbm,
                     idx_v, rows_v, gsems):
    k = _CHUNK // _IDX_PER_STREAM
    n_chunks = n_per_w // _CHUNK
    wid = lax.axis_index("s") * num_cores + lax.axis_index("c")
    base = wid * n_per_w
    base_g = wid * (n_per_w // _IDX_PER_STREAM)

    def load_idx(c, b):
        pltpu.sync_copy(tokens_hbm.at[pl.ds(base_g + c * k, k)], idx_v.at[b])

    def fire_gathers(b):
        for j in range(k):
            pltpu.async_copy(
                table_hbm.at[idx_v.at[b, j]],
                rows_v.at[b, pl.ds(j * _IDX_PER_STREAM, _IDX_PER_STREAM)],
                gsems[b],
            )

    def wait_gathers(b):
        for j in range(k):
            pltpu.make_async_copy(
                table_hbm.at[idx_v.at[b, j]],
                rows_v.at[b, pl.ds(j * _IDX_PER_STREAM, _IDX_PER_STREAM)],
                gsems[b],
            ).wait()

    def scale(b):
        @plsc.parallel_loop(0, _CHUNK, unroll=4)
        def _scale(r):
            for d in range(_EMB // _LANES):
                sl = (b, r, pl.ds(d * _LANES, _LANES))
                rows_v[sl] = rows_v[sl] * _SCALE

    def scatter(c, b):
        pltpu.sync_copy(rows_v.at[b], out_hbm.at[pl.ds(base + c * _CHUNK, _CHUNK)])

    for c in range(_NBUF):
        load_idx(c, c)
        fire_gathers(c)

    @pl.loop(0, n_chunks - _NBUF, step=_NBUF)
    def _main(i):
        for j in range(_NBUF):
            c = i + j
            wait_gathers(j)
            scale(j)
            scatter(c, j)
            load_idx(c + _NBUF, j)
            fire_gathers(j)

    for cc in range(n_chunks - _NBUF, n_chunks):
        b = cc % _NBUF
        wait_gathers(b)
        scale(b)
        scatter(cc, b)


def kernel(tokens, table):
    b, l = tokens.shape
    v, d = table.shape
    n = b * l
    info = plsc.get_sparse_core_info()
    nw = info.num_cores * info.num_subcores
    n_per_w = n // nw

    mesh = plsc.VectorSubcoreMesh(core_axis_name="c", subcore_axis_name="s")
    emb = pl.kernel(
        functools.partial(_emb_kernel_body, n_per_w, info.num_cores),
        out_type=jax.ShapeDtypeStruct((n, d), jnp.float32),
        mesh=mesh,
        scratch_types=[
            pltpu.VMEM((_NBUF, _CHUNK // _IDX_PER_STREAM, _IDX_PER_STREAM),
                       jnp.int32),
            pltpu.VMEM((_NBUF, _CHUNK, d), jnp.float32),
            [pltpu.SemaphoreType.DMA] * _NBUF,
        ],
        compiler_params=pltpu.CompilerParams(use_tc_tiling_on_sc=False),
    )
    flat = emb(jnp.reshape(tokens, (n // _IDX_PER_STREAM, _IDX_PER_STREAM)), table)
    return jnp.reshape(flat, (b, l, d))


# speedup vs baseline: 1.0843x; 1.0037x over previous
import functools
import math

import jax
import jax.numpy as jnp
from jax import lax
from jax.experimental import pallas as pl
from jax.experimental.pallas import tpu as pltpu
from jax.experimental.pallas import tpu_sc as plsc

_EMB = 64
_SCALE = math.sqrt(_EMB)
_LANES = 16
_IDX_PER_STREAM = 40
_B_PER_CHUNK = 4
_NBUF = 2


def _emb_kernel_body(seq_l, b_per_w, num_cores, tokens_hbm, table_hbm, out_hbm,
                     idx_v, rows_v, gsems):
    chunk = _B_PER_CHUNK * seq_l
    k = chunk // _IDX_PER_STREAM
    n_chunks = b_per_w // _B_PER_CHUNK
    spr = seq_l // _IDX_PER_STREAM
    wid = lax.axis_index("s") * num_cores + lax.axis_index("c")
    b_base = wid * b_per_w

    def load_idx(c, b):
        off = (b_base + c * _B_PER_CHUNK) * seq_l
        pltpu.sync_copy(tokens_hbm.at[pl.ds(off, chunk)], idx_v.at[b])

    def gather_descs(b):
        for j in range(k):
            yield (
                table_hbm.at[idx_v.at[b, pl.ds(j * _IDX_PER_STREAM, _IDX_PER_STREAM)]],
                rows_v.at[b, j // spr,
                          pl.ds((j % spr) * _IDX_PER_STREAM, _IDX_PER_STREAM)],
            )

    def fire_gathers(b):
        for src, dst in gather_descs(b):
            pltpu.async_copy(src, dst, gsems[b])

    def wait_gathers(b):
        for src, dst in gather_descs(b):
            pltpu.make_async_copy(src, dst, gsems[b]).wait()

    def scale(b):
        for i in range(_B_PER_CHUNK):
            @plsc.parallel_loop(0, seq_l, unroll=4)
            def _scale(r):
                for d in range(_EMB // _LANES):
                    sl = (b, i, r, pl.ds(d * _LANES, _LANES))
                    rows_v[sl] = rows_v[sl] * _SCALE

    def scatter(c, b):
        pltpu.sync_copy(rows_v.at[b],
                        out_hbm.at[pl.ds(b_base + c * _B_PER_CHUNK, _B_PER_CHUNK)])

    for c in range(_NBUF):
        load_idx(c, c)
        fire_gathers(c)

    @pl.loop(0, n_chunks - _NBUF, step=_NBUF)
    def _main(i):
        for j in range(_NBUF):
            c = i + j
            wait_gathers(j)
            scale(j)
            scatter(c, j)
            load_idx(c + _NBUF, j)
            fire_gathers(j)

    for cc in range(n_chunks - _NBUF, n_chunks):
        b = cc % _NBUF
        wait_gathers(b)
        scale(b)
        scatter(cc, b)


def kernel(tokens, table):
    b, l = tokens.shape
    v, d = table.shape
    n = b * l
    info = plsc.get_sparse_core_info()
    nw = info.num_cores * info.num_subcores
    b_per_w = b // nw

    mesh = plsc.VectorSubcoreMesh(core_axis_name="c", subcore_axis_name="s")
    emb = pl.kernel(
        functools.partial(_emb_kernel_body, l, b_per_w, info.num_cores),
        out_type=jax.ShapeDtypeStruct((b, l, d), jnp.float32),
        mesh=mesh,
        scratch_types=[
            pltpu.VMEM((_NBUF, _B_PER_CHUNK * l), jnp.int32),
            pltpu.VMEM((_NBUF, _B_PER_CHUNK, l, d), jnp.float32),
            [pltpu.SemaphoreType.DMA] * _NBUF,
        ],
        compiler_params=pltpu.CompilerParams(use_tc_tiling_on_sc=False),
    )
    return emb(jnp.reshape(tokens, (n,)), table)
